# baseline (speedup 1.0000x reference)
import dataclasses

import jax
import jax.numpy as jnp
from jax.experimental import pallas as pl
from jax.experimental.pallas import tpu as pltpu
from jax.experimental.pallas import tpu_sc as plsc


def _sc_compiler_params():
    cp = pltpu.CompilerParams()
    if "needs_layout_passes" in pltpu.CompilerParams.__dataclass_fields__:
        cp = dataclasses.replace(cp, needs_layout_passes=False)
    return cp

EPS_ = 1e-05
G_ = 64
D_ = 128
BR_ = 20000
CH_ = 512
NCH_ = -(-BR_ // CH_)
TAIL_ = BR_ - (NCH_ - 1) * CH_

SC_CORES_ = 2
SC_SUBCORES_ = 16
SC_WORKERS_ = SC_CORES_ * SC_SUBCORES_
SC_BINS_ = 128


def _chunk_onehot(ids_row):
    seg = jax.lax.broadcasted_iota(jnp.int32, (G_, CH_), 0)
    return (seg == ids_row).astype(jnp.float32)


def _stats_body(ids_ref, x_ref, acc_ref):
    i = pl.program_id(0)

    @pl.when(i == 0)
    def _init():
        acc_ref[...] = jnp.zeros_like(acc_ref)

    for c in range(NCH_):
        base = min(c * CH_, BR_ - CH_)
        ids_row = ids_ref[0, c, :].reshape(1, CH_)
        oh = _chunk_onehot(ids_row)
        x = x_ref[pl.ds(base, CH_), :]
        acc_ref[:, :D_] += jax.lax.dot_general(
            oh, x, (((1,), (0,)), ((), ())),
            preferred_element_type=jnp.float32)
        acc_ref[:, D_:] += jax.lax.dot_general(
            oh, x * x, (((1,), (0,)), ((), ())),
            preferred_element_type=jnp.float32)


SC_WIN_ = 512


def _sc_counts(ids_pad):
    n_pad = ids_pad.shape[1]

    @pl.kernel(
        out_type=jax.ShapeDtypeStruct((SC_WORKERS_, SC_BINS_), jnp.float32),
        mesh=plsc.VectorSubcoreMesh(core_axis_name="c", subcore_axis_name="s"),
        scratch_types=[
            pltpu.VMEM((1, SC_BINS_), jnp.float32),
        ],
        compiler_params=_sc_compiler_params(),
    )
    def counts_kernel(ids_hbm, o_hbm, hist_vmem):
        w = jax.lax.axis_index("c") * SC_SUBCORES_ + jax.lax.axis_index("s")

        @pl.loop(0, SC_BINS_ // 16)
        def _zero(j):
            hist_vmem[0, pl.ds(j * 16, 16)] = jnp.zeros((16,), jnp.float32)

        ones = jnp.full((16,), 1.0, jnp.float32)
        row0 = jnp.zeros((16,), jnp.int32)

        def window_body(i_vmem):
            @pl.loop(0, SC_WIN_ // 16)
            def _hist(j):
                idv = i_vmem[0, pl.ds(j * 16, 16)]
                plsc.addupdate_scatter(hist_vmem, [row0, idv], ones)

        pltpu.emit_pipeline(
            window_body,
            grid=(n_pad // SC_WIN_,),
            in_specs=[pl.BlockSpec((1, SC_WIN_), lambda i: (0, i))],
            out_specs=[],
            core_axis_name=("c", "s"),
            dimension_semantics=(pltpu.PARALLEL,),
        )(ids_hbm)

        pltpu.sync_copy(hist_vmem, o_hbm.at[pl.ds(w, 1)])

    return counts_kernel(ids_pad)[:, :G_]


def _apply_body(ids_ref, x_ref, cnt_ref, ms_ref, w_ref, b_ref, acc_ref,
                o_ref, params_ref):
    i = pl.program_id(0)

    @pl.when(i == 0)
    def _finalize():
        counts = jnp.maximum(jnp.sum(cnt_ref[...], axis=0), 1.0)
        mean = acc_ref[:, :D_] / counts[:, None]
        m2 = acc_ref[:, D_:] / counts[:, None]
        ms = ms_ref[...]
        a = mean * ms
        var = m2 - 2.0 * a * mean + a * a
        s = w_ref[...] * jax.lax.rsqrt(var + EPS_)
        params_ref[:, :D_] = s
        params_ref[:, D_:] = a * s

    b = b_ref[...]
    for c in range(NCH_):
        base = min(c * CH_, BR_ - CH_)
        ids_row = ids_ref[0, c, :].reshape(1, CH_)
        oh = _chunk_onehot(ids_row)
        x = x_ref[pl.ds(base, CH_), :]
        g = jax.lax.dot_general(
            oh, params_ref[...], (((0,), (0,)), ((), ())),
            preferred_element_type=jnp.float32)
        y = x * g[:, :D_] - g[:, D_:] + b
        if c < NCH_ - 1:
            o_ref[pl.ds(base, CH_), :] = y
        else:
            skip = CH_ - TAIL_
            o_ref[pl.ds(base + skip, TAIL_), :] = y[skip:, :]


def kernel(features, weight, bias, mean_scale, segment_ids, num_segments):
    n, d = features.shape
    assert d == D_ and n % BR_ == 0
    nb = n // BR_
    ids = segment_ids.astype(jnp.int32)
    ids2d = ids.reshape(nb, BR_)
    head = ids2d[:, :(NCH_ - 1) * CH_].reshape(nb, NCH_ - 1, CH_)
    tail = jnp.concatenate(
        [jnp.full((nb, 1, CH_ - TAIL_), -1, jnp.int32),
         ids2d[:, BR_ - TAIL_:].reshape(nb, 1, TAIL_)], axis=2)
    ids_chunks = jnp.concatenate([head, tail], axis=1)
    ms = mean_scale.reshape(1, D_)
    w = weight.reshape(1, D_)
    b = bias.reshape(1, D_)

    n_pad = -(-n // SC_WIN_) * SC_WIN_
    ids_pad = jnp.concatenate(
        [ids, jnp.full((n_pad - n,), G_, jnp.int32)]).reshape(1, n_pad)

    acc = pl.pallas_call(
        _stats_body,
        grid=(nb,),
        in_specs=[
            pl.BlockSpec((1, NCH_, CH_), lambda i: (i, 0, 0)),
            pl.BlockSpec((BR_, D_), lambda i: (i, 0)),
        ],
        out_specs=pl.BlockSpec((G_, 2 * D_), lambda i: (0, 0)),
        out_shape=jax.ShapeDtypeStruct((G_, 2 * D_), jnp.float32),
        compiler_params=pltpu.CompilerParams(
            dimension_semantics=("arbitrary",)),
    )(ids_chunks, features)

    cnt_parts = _sc_counts(ids_pad)

    out = pl.pallas_call(
        _apply_body,
        grid=(nb,),
        in_specs=[
            pl.BlockSpec((1, NCH_, CH_), lambda i: (i, 0, 0)),
            pl.BlockSpec((BR_, D_), lambda i: (i, 0)),
            pl.BlockSpec((SC_WORKERS_, G_), lambda i: (0, 0)),
            pl.BlockSpec((1, D_), lambda i: (0, 0)),
            pl.BlockSpec((1, D_), lambda i: (0, 0)),
            pl.BlockSpec((1, D_), lambda i: (0, 0)),
            pl.BlockSpec((G_, 2 * D_), lambda i: (0, 0)),
        ],
        out_specs=pl.BlockSpec((BR_, D_), lambda i: (i, 0)),
        out_shape=jax.ShapeDtypeStruct((n, D_), jnp.float32),
        scratch_shapes=[
            pltpu.VMEM((G_, 2 * D_), jnp.float32),
        ],
        compiler_params=pltpu.CompilerParams(
            dimension_semantics=("arbitrary",)),
    )(ids_chunks, features, cnt_parts, ms, w, b, acc)
    return out


# speedup vs baseline: 1.0012x; 1.0012x over previous
import dataclasses

import jax
import jax.numpy as jnp
from jax.experimental import pallas as pl
from jax.experimental.pallas import tpu as pltpu
from jax.experimental.pallas import tpu_sc as plsc


def _sc_compiler_params():
    cp = pltpu.CompilerParams()
    if "needs_layout_passes" in pltpu.CompilerParams.__dataclass_fields__:
        cp = dataclasses.replace(cp, needs_layout_passes=False)
    return cp

EPS_ = 1e-05
G_ = 64
D_ = 128
BR_ = 20000
CH_ = 512
NCH_ = -(-BR_ // CH_)
TAIL_ = BR_ - (NCH_ - 1) * CH_

SC_CORES_ = 2
SC_SUBCORES_ = 16
SC_WORKERS_ = SC_CORES_ * SC_SUBCORES_
SC_BINS_ = 128


def _chunk_onehot(ids_row):
    seg = jax.lax.broadcasted_iota(jnp.int32, (G_, CH_), 0)
    return (seg == ids_row).astype(jnp.float32)


def _stats_body(ids_ref, x_ref, acc_ref):
    i = pl.program_id(0)

    @pl.when(i == 0)
    def _init():
        acc_ref[...] = jnp.zeros_like(acc_ref)

    for c in range(NCH_):
        base = min(c * CH_, BR_ - CH_)
        ids_row = ids_ref[0, c, :].reshape(1, CH_)
        oh = _chunk_onehot(ids_row)
        x = x_ref[pl.ds(base, CH_), :]
        acc_ref[:, :D_] += jax.lax.dot_general(
            oh, x, (((1,), (0,)), ((), ())),
            preferred_element_type=jnp.float32)
        acc_ref[:, D_:] += jax.lax.dot_general(
            oh, x * x, (((1,), (0,)), ((), ())),
            preferred_element_type=jnp.float32)


SC_WIN_ = 3200


def _sc_counts(ids_pad):
    n_pad = ids_pad.shape[1]

    @pl.kernel(
        out_type=jax.ShapeDtypeStruct((SC_WORKERS_, SC_BINS_), jnp.float32),
        mesh=plsc.VectorSubcoreMesh(core_axis_name="c", subcore_axis_name="s"),
        scratch_types=[
            pltpu.VMEM((1, SC_BINS_), jnp.float32),
        ],
        compiler_params=_sc_compiler_params(),
    )
    def counts_kernel(ids_hbm, o_hbm, hist_vmem):
        w = jax.lax.axis_index("c") * SC_SUBCORES_ + jax.lax.axis_index("s")

        @pl.loop(0, SC_BINS_ // 16)
        def _zero(j):
            hist_vmem[0, pl.ds(j * 16, 16)] = jnp.zeros((16,), jnp.float32)

        ones = jnp.full((16,), 1.0, jnp.float32)
        row0 = jnp.zeros((16,), jnp.int32)

        def window_body(i_vmem):
            @pl.loop(0, SC_WIN_ // 16)
            def _hist(j):
                idv = i_vmem[0, pl.ds(j * 16, 16)]
                plsc.addupdate_scatter(hist_vmem, [row0, idv], ones)

        pltpu.emit_pipeline(
            window_body,
            grid=(n_pad // SC_WIN_,),
            in_specs=[pl.BlockSpec((1, SC_WIN_), lambda i: (0, i))],
            out_specs=[],
            core_axis_name=("c", "s"),
            dimension_semantics=(pltpu.PARALLEL,),
        )(ids_hbm)

        pltpu.sync_copy(hist_vmem, o_hbm.at[pl.ds(w, 1)])

    return counts_kernel(ids_pad)[:, :G_]


def _apply_body(ids_ref, x_ref, cnt_ref, ms_ref, w_ref, b_ref, acc_ref,
                o_ref, params_ref):
    i = pl.program_id(0)

    @pl.when(i == 0)
    def _finalize():
        counts = jnp.maximum(jnp.sum(cnt_ref[...], axis=0), 1.0)
        mean = acc_ref[:, :D_] / counts[:, None]
        m2 = acc_ref[:, D_:] / counts[:, None]
        ms = ms_ref[...]
        a = mean * ms
        var = m2 - 2.0 * a * mean + a * a
        s = w_ref[...] * jax.lax.rsqrt(var + EPS_)
        params_ref[:, :D_] = s
        params_ref[:, D_:] = a * s

    b = b_ref[...]
    for c in range(NCH_):
        base = min(c * CH_, BR_ - CH_)
        ids_row = ids_ref[0, c, :].reshape(1, CH_)
        oh = _chunk_onehot(ids_row)
        x = x_ref[pl.ds(base, CH_), :]
        g = jax.lax.dot_general(
            oh, params_ref[...], (((0,), (0,)), ((), ())),
            preferred_element_type=jnp.float32)
        y = x * g[:, :D_] - g[:, D_:] + b
        if c < NCH_ - 1:
            o_ref[pl.ds(base, CH_), :] = y
        else:
            skip = CH_ - TAIL_
            o_ref[pl.ds(base + skip, TAIL_), :] = y[skip:, :]


def kernel(features, weight, bias, mean_scale, segment_ids, num_segments):
    n, d = features.shape
    assert d == D_ and n % BR_ == 0
    nb = n // BR_
    ids = segment_ids.astype(jnp.int32)
    ids2d = ids.reshape(nb, BR_)
    head = ids2d[:, :(NCH_ - 1) * CH_].reshape(nb, NCH_ - 1, CH_)
    tail = jnp.concatenate(
        [jnp.full((nb, 1, CH_ - TAIL_), -1, jnp.int32),
         ids2d[:, BR_ - TAIL_:].reshape(nb, 1, TAIL_)], axis=2)
    ids_chunks = jnp.concatenate([head, tail], axis=1)
    ms = mean_scale.reshape(1, D_)
    w = weight.reshape(1, D_)
    b = bias.reshape(1, D_)

    n_pad = -(-n // SC_WIN_) * SC_WIN_
    ids_pad = jnp.concatenate(
        [ids, jnp.full((n_pad - n,), G_, jnp.int32)]).reshape(1, n_pad)

    acc = pl.pallas_call(
        _stats_body,
        grid=(nb,),
        in_specs=[
            pl.BlockSpec((1, NCH_, CH_), lambda i: (i, 0, 0)),
            pl.BlockSpec((BR_, D_), lambda i: (i, 0)),
        ],
        out_specs=pl.BlockSpec((G_, 2 * D_), lambda i: (0, 0)),
        out_shape=jax.ShapeDtypeStruct((G_, 2 * D_), jnp.float32),
        compiler_params=pltpu.CompilerParams(
            dimension_semantics=("arbitrary",)),
    )(ids_chunks, features)

    cnt_parts = _sc_counts(ids_pad)

    out = pl.pallas_call(
        _apply_body,
        grid=(nb,),
        in_specs=[
            pl.BlockSpec((1, NCH_, CH_), lambda i: (i, 0, 0)),
            pl.BlockSpec((BR_, D_), lambda i: (i, 0)),
            pl.BlockSpec((SC_WORKERS_, G_), lambda i: (0, 0)),
            pl.BlockSpec((1, D_), lambda i: (0, 0)),
            pl.BlockSpec((1, D_), lambda i: (0, 0)),
            pl.BlockSpec((1, D_), lambda i: (0, 0)),
            pl.BlockSpec((G_, 2 * D_), lambda i: (0, 0)),
        ],
        out_specs=pl.BlockSpec((BR_, D_), lambda i: (i, 0)),
        out_shape=jax.ShapeDtypeStruct((n, D_), jnp.float32),
        scratch_shapes=[
            pltpu.VMEM((G_, 2 * D_), jnp.float32),
        ],
        compiler_params=pltpu.CompilerParams(
            dimension_semantics=("arbitrary",)),
    )(ids_chunks, features, cnt_parts, ms, w, b, acc)
    return out
